# baseline (speedup 1.0000x reference)
import functools

import jax
import jax.numpy as jnp
from jax import lax
from jax.experimental import pallas as pl
from jax.experimental.pallas import tpu as pltpu
from jax.experimental.pallas import tpu_sc as plsc

VOCAB = 1000000
EMBED = 64
OUT = 100000
B = 1024
L = 200

NC = 2
NS = 16
NW = NC * NS
BPW = B // NW
IPW = BPW * L
CH1, CH2 = 128, 72


def _sc_pool_body(idx_hbm, table_hbm, out_hbm, idx_v, rows_v, acc_v, sem):
    wid = lax.axis_index("s") * NC + lax.axis_index("c")
    pltpu.sync_copy(idx_hbm.at[wid], idx_v)
    for i in range(BPW):
        off = i * L
        g1 = pltpu.async_copy(
            table_hbm.at[idx_v.at[pl.ds(off, CH1)]],
            rows_v.at[pl.ds(0, CH1)], sem)
        g2 = pltpu.async_copy(
            table_hbm.at[idx_v.at[pl.ds(off + CH1, CH2)]],
            rows_v.at[pl.ds(CH1, CH2)], sem)
        g1.wait()
        g2.wait()

        def body(j, carry):
            a0, a1, a2, a3 = carry
            a0 = a0 + rows_v[j, pl.ds(0, 16)]
            a1 = a1 + rows_v[j, pl.ds(16, 16)]
            a2 = a2 + rows_v[j, pl.ds(32, 16)]
            a3 = a3 + rows_v[j, pl.ds(48, 16)]
            return a0, a1, a2, a3

        z = jnp.zeros((16,), jnp.float32)
        a0, a1, a2, a3 = lax.fori_loop(0, L, body, (z, z, z, z))
        acc_v[i, pl.ds(0, 16)] = a0
        acc_v[i, pl.ds(16, 16)] = a1
        acc_v[i, pl.ds(32, 16)] = a2
        acc_v[i, pl.ds(48, 16)] = a3
    pltpu.sync_copy(acc_v, out_hbm.at[pl.ds(wid * BPW, BPW)])


_sc_pool = functools.partial(
    pl.kernel,
    mesh=plsc.VectorSubcoreMesh(core_axis_name="c", subcore_axis_name="s"),
    out_type=jax.ShapeDtypeStruct((B, EMBED), jnp.float32),
    scratch_types=[
        pltpu.VMEM((IPW,), jnp.int32),
        pltpu.VMEM((L, EMBED), jnp.float32),
        pltpu.VMEM((BPW, EMBED), jnp.float32),
        pltpu.SemaphoreType.DMA,
    ],
    compiler_params=pltpu.CompilerParams(use_tc_tiling_on_sc=False),
)(_sc_pool_body)


BLK = 2048
NBLK = (OUT + BLK - 1) // BLK


def _mm_body(p_ref, w_ref, b_ref, o_ref):
    o_ref[:] = lax.dot_general(
        p_ref[:], w_ref[:], (((1,), (1,)), ((), ())),
        preferred_element_type=jnp.float32) + b_ref[:]


def _matmul(pooled, W, b2):
    return pl.pallas_call(
        _mm_body,
        grid=(NBLK,),
        in_specs=[
            pl.BlockSpec((B, EMBED), lambda j: (0, 0)),
            pl.BlockSpec((BLK, EMBED), lambda j: (j, 0)),
            pl.BlockSpec((1, BLK), lambda j: (0, j)),
        ],
        out_specs=pl.BlockSpec((B, BLK), lambda j: (0, j)),
        out_shape=jax.ShapeDtypeStruct((B, OUT), jnp.float32),
    )(pooled, W, b2)


def kernel(inputs, table, W, b):
    idx = inputs.astype(jnp.int32).reshape(NW, IPW)
    pooled = _sc_pool(idx, table)
    return _matmul(pooled, W, b.reshape(1, OUT))
